# baseline (speedup 1.0000x reference)
import jax
import jax.numpy as jnp
import numpy as np
from jax.experimental import pallas as pl
from jax.experimental.pallas import tpu as pltpu

B, N = 8, 128
NF = 128
RW = 16
DMAX = 64
HE = 64
DN = NF + HE
TD = 4 * NF
L = 3
P = 16
NB = N // P
GF = 32

_F32 = jnp.float32


def _silu(v):
    return v * jax.nn.sigmoid(v)


def _mm(a, b):
    return jnp.dot(a, b, preferred_element_type=_F32,
                   precision=jax.lax.Precision.HIGHEST)


def _mmx(a, b):
    return jnp.dot(a, b, preferred_element_type=_F32,
                   precision=jax.lax.Precision.HIGHEST)


def _kernel_body(x_ref, t_ref, mask_ref,
                 Wt1_ref, bt1_ref, Wt2_ref, bt2_ref,
                 WoriT_ref, eb_ref, WspdT_ref,
                 Wdeg_ref, bdeg_ref, Wpos_ref, bpos_ref,
                 Ws_ref, Wd_ref, We_ref, Wu_ref, bu_ref, Wtb_ref,
                 Wpi_ref, Wpj_ref, Wpe_ref, bp_ref,
                 Wo1_ref, bo1_ref, Wo2_ref, bo2_ref,
                 out_ref,
                 eattr_ref, v_ref, s_ref, b_ref, acc_ref, a_ref, c_ref,
                 x_s, sp_s):
    x_raw = x_ref[0, 0]
    m_raw = mask_ref[0, 0]
    xc = jnp.clip(jnp.where(jnp.isnan(x_raw), 0.0, x_raw), -1.0, 1.0)
    mask_c = jnp.clip(jnp.where(jnp.isnan(m_raw), 0.0, m_raw), 0.0, 1.0)
    cont_adj = jnp.clip((xc + 1.0) * 0.5 * mask_c, 0.0, 1.0)
    adj = (xc >= 0.0).astype(_F32) * mask_c
    v_ref[...] = (cont_adj > 0.0).astype(_F32)

    t = t_ref[0, 0, 0]
    half = NF // 2
    freqs = jnp.exp(
        jax.lax.broadcasted_iota(jnp.int32, (1, half), 1).astype(_F32)
        * (-np.log(10000.0) / (half - 1)))
    ang = t * freqs * 1000.0
    temb = jnp.concatenate([jnp.sin(ang), jnp.cos(ang)], axis=1)
    temb = _mm(temb, Wt1_ref[...]) + bt1_ref[...]
    temb = _mm(_silu(temb), Wt2_ref[...]) + bt2_ref[...]
    temb_a = _silu(temb)

    eye = (jax.lax.broadcasted_iota(jnp.int32, (N, N), 0)
           == jax.lax.broadcasted_iota(jnp.int32, (N, N), 1))
    eyef = eye.astype(_F32)
    deg_b = jnp.sum(adj, axis=1, keepdims=True)
    T = adj / jnp.maximum(deg_b, 1.0)

    Pk = T
    Akf = (adj > 0.0).astype(_F32)
    reached = eyef
    spd = jnp.where(eye, 0.0, float(RW))
    hpos = jnp.zeros((N, HE), _F32)
    for k in range(1, RW + 1):
        d = jnp.sum(Pk * eyef, axis=1, keepdims=True)
        hpos = hpos + d * Wpos_ref[k - 1:k, :]
        newf = Akf * (1.0 - reached)
        spd = jnp.where(newf > 0.0, float(k), spd)
        reached = reached + newf
        if k < RW:
            Pk = _mmx(Pk, T)
            Akf = (_mm(Akf, adj) > 0.0).astype(_F32)
    hpos = hpos + bpos_ref[...]

    degc = jnp.clip(jnp.sum(cont_adj, axis=1, keepdims=True), 0.0, float(DMAX))
    degi = jnp.floor(degc)
    iota_d = jax.lax.broadcasted_iota(jnp.int32, (N, DMAX + 1), 1).astype(_F32)
    oh = (degi == iota_d).astype(_F32)
    hdeg = _mm(oh, Wdeg_ref[...]) + bdeg_ref[...]
    h = jnp.concatenate([hdeg, hpos], axis=1)

    wori = WoriT_ref[...]
    ebias = eb_ref[...]
    Wcat = jnp.concatenate(
        [wori, WspdT_ref[...], ebias, jnp.zeros((GF - RW - 3, HE), _F32)],
        axis=0)

    x_s[...] = xc
    sp_s[...] = spd

    def g_block(p, _):
        xb = x_s[pl.ds(p * P, P), :]
        sb = sp_s[pl.ds(p * P, P), :]
        vb = v_ref[pl.ds(p * P, P), :]
        iota = jax.lax.broadcasted_iota(jnp.int32, (P, N, GF), 2)
        g = (sb[:, :, None] == (iota - 1).astype(_F32)).astype(_F32)
        g = g + jnp.where(iota == 0, xb[:, :, None], 0.0)
        g = g + (iota == RW + 2).astype(_F32)
        g = g + (iota == RW + 3).astype(_F32) * (1.0 - vb[:, :, None])
        eattr_ref[pl.ds(p * P * N, P * N), :] = g.reshape(P * N, GF)
        return 0

    jax.lax.fori_loop(0, NB, g_block, 0)
    mrow = jax.lax.broadcasted_iota(jnp.int32, (GF, DN), 0)

    for l in range(L):
        s_ref[...] = _mm(h, Ws_ref[l])
        b_ref[...] = _mm(h, Wd_ref[l]) + _mm(temb_a, Wtb_ref[l])
        acc_ref[...] = jnp.zeros((N, DN), _F32)
        Ml = jnp.where(mrow == RW + 3, -1e5, _mmx(Wcat, We_ref[l]))

        def msg_block(p, _, Ml=Ml):
            gb = eattr_ref[pl.ds(p * P * N, P * N), :]
            ew = _mm(gb, Ml).reshape(P, N, DN)
            Sb = s_ref[pl.ds(p * P, P), :]
            msg = _silu(ew + Sb[:, None, :] + b_ref[...][None])
            acc_ref[...] += jnp.sum(msg, axis=0)
            return 0

        jax.lax.fori_loop(0, NB, msg_block, 0)
        h = _silu(h + _mm(acc_ref[...], Wu_ref[l]) + bu_ref[l:l + 1, :])

    a_ref[...] = _mm(h, Wpi_ref[...])
    c_ref[...] = _mm(h, Wpj_ref[...])
    bp = bp_ref[...]
    wo2 = Wo2_ref[...]
    Mpe = _mmx(Wcat, Wpe_ref[...])
    Wo1 = Wo1_ref[...]
    bo1 = bo1_ref[...]
    bo2 = bo2_ref[0, 0]

    def head_block(p, _):
        gb = eattr_ref[pl.ds(p * P * N, P * N), :]
        ep = _mm(gb, Mpe).reshape(P, N, HE)
        ab = a_ref[pl.ds(p * P, P), :]
        ed = _silu(ep + ab[:, None, :] + c_ref[...][None] + bp[None])
        v = _silu(ed).reshape(P * N, HE)
        v2 = jax.lax.dot_general(v, Wo1, (((1,), (1,)), ((), ())),
                                 preferred_element_type=_F32) + bo1
        v2 = _silu(v2).reshape(P, N, HE)
        o = jnp.sum(v2 * wo2[None], axis=2) + bo2
        out_ref[0, 0, pl.ds(p * P, P), :] = o
        return 0

    jax.lax.fori_loop(0, NB, head_block, 0)
    out = out_ref[0, 0]
    out_ref[0, 0] = (out + out.T) * 0.5 * mask_c


def _full_spec(shape):
    nd = len(shape)
    return pl.BlockSpec(shape, lambda b: (0,) * nd)


@jax.jit
def kernel(x, time_cond, mask, Wt1, bt1, Wt2, bt2, Wori, bori, Wspd, bspd,
           Wdeg, bdeg, Wpos, bpos, Ws, Wd, We, Wu, bu, Wtb, Wpi, Wpj, Wpe,
           bp, Wo1, bo1, Wo2, bo2):
    t2 = time_cond.reshape(B, 1, 1).astype(_F32)
    args = (
        x, t2, mask,
        Wt1, bt1.reshape(1, TD), Wt2, bt2.reshape(1, TD),
        Wori.T, (bori + bspd).reshape(1, HE), Wspd.T,
        Wdeg, bdeg.reshape(1, NF), Wpos, bpos.reshape(1, HE),
        Ws, Wd, We, Wu, bu, Wtb,
        Wpi, Wpj, Wpe, bp.reshape(1, HE),
        Wo1, bo1.reshape(1, HE), Wo2, bo2.reshape(1, 1),
    )
    in_specs = [
        pl.BlockSpec((1, 1, N, N), lambda b: (b, 0, 0, 0)),
        pl.BlockSpec((1, 1, 1), lambda b: (b, 0, 0)),
        pl.BlockSpec((1, 1, N, N), lambda b: (b, 0, 0, 0)),
    ] + [_full_spec(a.shape) for a in args[3:]]
    out = pl.pallas_call(
        _kernel_body,
        grid=(B,),
        in_specs=in_specs,
        out_specs=pl.BlockSpec((1, 1, N, N), lambda b: (b, 0, 0, 0)),
        out_shape=jax.ShapeDtypeStruct((B, 1, N, N), _F32),
        scratch_shapes=[
            pltpu.VMEM((N * N, GF), _F32),
            pltpu.VMEM((N, N), _F32),
            pltpu.VMEM((N, DN), _F32),
            pltpu.VMEM((N, DN), _F32),
            pltpu.VMEM((N, DN), _F32),
            pltpu.VMEM((N, HE), _F32),
            pltpu.VMEM((N, HE), _F32),
            pltpu.VMEM((N, N), _F32),
            pltpu.VMEM((N, N), _F32),
        ],
        compiler_params=pltpu.CompilerParams(
            dimension_semantics=("parallel",)),
    )(*args)
    return out


# speedup vs baseline: 2.1089x; 2.1089x over previous
import jax
import jax.numpy as jnp
import numpy as np
from jax.experimental import pallas as pl
from jax.experimental.pallas import tpu as pltpu

B, N = 8, 128
NF = 128
RW = 16
DMAX = 64
HE = 64
DN = NF + HE
TD = 4 * NF
L = 3
P = 16
NB = N // P
GF = 32

_F32 = jnp.float32


def _silu(v):
    return v * (0.5 * jnp.tanh(0.5 * v) + 0.5)


def _mm(a, b):
    return jnp.dot(a, b, preferred_element_type=_F32)


_mmx = _mm


def _kernel_body(x_ref, t_ref, mask_ref,
                 Wt1_ref, bt1_ref, Wt2_ref, bt2_ref,
                 WoriT_ref, eb_ref, WspdT_ref,
                 Wdeg_ref, bdeg_ref, Wpos_ref, bpos_ref,
                 Ws_ref, Wd_ref, We_ref, Wu_ref, bu_ref, Wtb_ref,
                 Wpi_ref, Wpj_ref, Wpe_ref, bp_ref,
                 Wo1_ref, bo1_ref, Wo2_ref, bo2_ref,
                 out_ref,
                 eattr_ref, v_ref, s_ref, b_ref, acc_ref, a_ref, c_ref,
                 x_s, sp_s):
    x_raw = x_ref[0, 0]
    m_raw = mask_ref[0, 0]
    xc = jnp.clip(jnp.where(jnp.isnan(x_raw), 0.0, x_raw), -1.0, 1.0)
    mask_c = jnp.clip(jnp.where(jnp.isnan(m_raw), 0.0, m_raw), 0.0, 1.0)
    cont_adj = jnp.clip((xc + 1.0) * 0.5 * mask_c, 0.0, 1.0)
    adj = (xc >= 0.0).astype(_F32) * mask_c
    v_ref[...] = (cont_adj > 0.0).astype(_F32)

    t = t_ref[0, 0, 0]
    half = NF // 2
    freqs = jnp.exp(
        jax.lax.broadcasted_iota(jnp.int32, (1, half), 1).astype(_F32)
        * (-np.log(10000.0) / (half - 1)))
    ang = t * freqs * 1000.0
    temb = jnp.concatenate([jnp.sin(ang), jnp.cos(ang)], axis=1)
    temb = _mm(temb, Wt1_ref[...]) + bt1_ref[...]
    temb = _mm(_silu(temb), Wt2_ref[...]) + bt2_ref[...]
    temb_a = _silu(temb)

    eye = (jax.lax.broadcasted_iota(jnp.int32, (N, N), 0)
           == jax.lax.broadcasted_iota(jnp.int32, (N, N), 1))
    eyef = eye.astype(_F32)
    deg_b = jnp.sum(adj, axis=1, keepdims=True)
    T = adj / jnp.maximum(deg_b, 1.0)

    Pk = T
    Akf = (adj > 0.0).astype(_F32)
    reached = eyef
    spd = jnp.where(eye, 0.0, float(RW))
    hpos = jnp.zeros((N, HE), _F32)
    for k in range(1, RW + 1):
        d = jnp.sum(Pk * eyef, axis=1, keepdims=True)
        hpos = hpos + d * Wpos_ref[k - 1:k, :]
        newf = Akf * (1.0 - reached)
        spd = jnp.where(newf > 0.0, float(k), spd)
        reached = reached + newf
        if k < RW:
            Pk = _mmx(Pk, T)
            Akf = (_mm(Akf, adj) > 0.0).astype(_F32)
    hpos = hpos + bpos_ref[...]

    degc = jnp.clip(jnp.sum(cont_adj, axis=1, keepdims=True), 0.0, float(DMAX))
    degi = jnp.floor(degc)
    iota_d = jax.lax.broadcasted_iota(jnp.int32, (N, DMAX + 1), 1).astype(_F32)
    oh = (degi == iota_d).astype(_F32)
    hdeg = _mm(oh, Wdeg_ref[...]) + bdeg_ref[...]
    h = jnp.concatenate([hdeg, hpos], axis=1)

    wori = WoriT_ref[...]
    ebias = eb_ref[...]
    Wcat = jnp.concatenate(
        [wori, WspdT_ref[...], ebias, jnp.zeros((GF - RW - 3, HE), _F32)],
        axis=0)

    x_s[...] = xc
    sp_s[...] = spd

    def g_block(p, _):
        xb = x_s[pl.ds(p * P, P), :]
        sb = sp_s[pl.ds(p * P, P), :]
        vb = v_ref[pl.ds(p * P, P), :]
        iota = jax.lax.broadcasted_iota(jnp.int32, (P, N, GF), 2)
        g = (sb[:, :, None] == (iota - 1).astype(_F32)).astype(_F32)
        g = g + jnp.where(iota == 0, xb[:, :, None], 0.0)
        g = g + (iota == RW + 2).astype(_F32)
        g = g + (iota == RW + 3).astype(_F32) * (1.0 - vb[:, :, None])
        eattr_ref[pl.ds(p * P * N, P * N), :] = g.reshape(P * N, GF)
        return 0

    jax.lax.fori_loop(0, NB, g_block, 0)
    mrow = jax.lax.broadcasted_iota(jnp.int32, (GF, DN), 0)

    for l in range(L):
        s_ref[...] = _mm(h, Ws_ref[l])
        b_ref[...] = _mm(h, Wd_ref[l]) + _mm(temb_a, Wtb_ref[l])
        acc_ref[...] = jnp.zeros((N, DN), _F32)
        Ml = jnp.where(mrow == RW + 3, -1e5, _mmx(Wcat, We_ref[l]))

        def msg_block(p, _, Ml=Ml):
            gb = eattr_ref[pl.ds(p * P * N, P * N), :]
            ew = _mm(gb, Ml).reshape(P, N, DN)
            Sb = s_ref[pl.ds(p * P, P), :]
            msg = _silu(ew + Sb[:, None, :] + b_ref[...][None])
            acc_ref[...] += jnp.sum(msg, axis=0)
            return 0

        jax.lax.fori_loop(0, NB, msg_block, 0)
        h = _silu(h + _mm(acc_ref[...], Wu_ref[l]) + bu_ref[l:l + 1, :])

    a_ref[...] = _mm(h, Wpi_ref[...])
    c_ref[...] = _mm(h, Wpj_ref[...])
    bp = bp_ref[...]
    wo2 = Wo2_ref[...]
    Mpe = _mmx(Wcat, Wpe_ref[...])
    Wo1 = Wo1_ref[...]
    bo1 = bo1_ref[...]
    bo2 = bo2_ref[0, 0]

    def head_block(p, _):
        gb = eattr_ref[pl.ds(p * P * N, P * N), :]
        ep = _mm(gb, Mpe).reshape(P, N, HE)
        ab = a_ref[pl.ds(p * P, P), :]
        ed = _silu(ep + ab[:, None, :] + c_ref[...][None] + bp[None])
        v = _silu(ed).reshape(P * N, HE)
        v2 = jax.lax.dot_general(v, Wo1, (((1,), (1,)), ((), ())),
                                 preferred_element_type=_F32) + bo1
        v2 = _silu(v2).reshape(P, N, HE)
        o = jnp.sum(v2 * wo2[None], axis=2) + bo2
        out_ref[0, 0, pl.ds(p * P, P), :] = o
        return 0

    jax.lax.fori_loop(0, NB, head_block, 0)
    out = out_ref[0, 0]
    out_ref[0, 0] = (out + out.T) * 0.5 * mask_c


def _full_spec(shape):
    nd = len(shape)
    return pl.BlockSpec(shape, lambda b: (0,) * nd)


@jax.jit
def kernel(x, time_cond, mask, Wt1, bt1, Wt2, bt2, Wori, bori, Wspd, bspd,
           Wdeg, bdeg, Wpos, bpos, Ws, Wd, We, Wu, bu, Wtb, Wpi, Wpj, Wpe,
           bp, Wo1, bo1, Wo2, bo2):
    t2 = time_cond.reshape(B, 1, 1).astype(_F32)
    args = (
        x, t2, mask,
        Wt1, bt1.reshape(1, TD), Wt2, bt2.reshape(1, TD),
        Wori.T, (bori + bspd).reshape(1, HE), Wspd.T,
        Wdeg, bdeg.reshape(1, NF), Wpos, bpos.reshape(1, HE),
        Ws, Wd, We, Wu, bu, Wtb,
        Wpi, Wpj, Wpe, bp.reshape(1, HE),
        Wo1, bo1.reshape(1, HE), Wo2, bo2.reshape(1, 1),
    )
    in_specs = [
        pl.BlockSpec((1, 1, N, N), lambda b: (b, 0, 0, 0)),
        pl.BlockSpec((1, 1, 1), lambda b: (b, 0, 0)),
        pl.BlockSpec((1, 1, N, N), lambda b: (b, 0, 0, 0)),
    ] + [_full_spec(a.shape) for a in args[3:]]
    out = pl.pallas_call(
        _kernel_body,
        grid=(B,),
        in_specs=in_specs,
        out_specs=pl.BlockSpec((1, 1, N, N), lambda b: (b, 0, 0, 0)),
        out_shape=jax.ShapeDtypeStruct((B, 1, N, N), _F32),
        scratch_shapes=[
            pltpu.VMEM((N * N, GF), _F32),
            pltpu.VMEM((N, N), _F32),
            pltpu.VMEM((N, DN), _F32),
            pltpu.VMEM((N, DN), _F32),
            pltpu.VMEM((N, DN), _F32),
            pltpu.VMEM((N, HE), _F32),
            pltpu.VMEM((N, HE), _F32),
            pltpu.VMEM((N, N), _F32),
            pltpu.VMEM((N, N), _F32),
        ],
        compiler_params=pltpu.CompilerParams(
            dimension_semantics=("parallel",)),
    )(*args)
    return out


# speedup vs baseline: 2.2911x; 1.0864x over previous
import jax
import jax.numpy as jnp
import numpy as np
from jax.experimental import pallas as pl
from jax.experimental.pallas import tpu as pltpu

B, N = 8, 128
NF = 128
RW = 16
DMAX = 64
HE = 64
DN = NF + HE
TD = 4 * NF
L = 3
P = 32
NB = N // P
GF = 32

_F32 = jnp.float32


def _silu(v):
    return v * (0.5 * jnp.tanh(0.5 * v) + 0.5)


def _mm(a, b):
    return jnp.dot(a, b, preferred_element_type=_F32)


_mmx = _mm


def _kernel_body(x_ref, t_ref, mask_ref,
                 Wt1_ref, bt1_ref, Wt2_ref, bt2_ref,
                 WoriT_ref, eb_ref, WspdT_ref,
                 Wdeg_ref, bdeg_ref, Wpos_ref, bpos_ref,
                 Ws_ref, Wd_ref, We_ref, Wu_ref, bu_ref, Wtb_ref,
                 Wpi_ref, Wpj_ref, Wpe_ref, bp_ref,
                 Wo1_ref, bo1_ref, Wo2_ref, bo2_ref,
                 out_ref,
                 eattr_ref, v_ref, s_ref, b_ref, acc_ref, a_ref, c_ref,
                 x_s, sp_s):
    x_raw = x_ref[0, 0]
    m_raw = mask_ref[0, 0]
    xc = jnp.clip(jnp.where(jnp.isnan(x_raw), 0.0, x_raw), -1.0, 1.0)
    mask_c = jnp.clip(jnp.where(jnp.isnan(m_raw), 0.0, m_raw), 0.0, 1.0)
    cont_adj = jnp.clip((xc + 1.0) * 0.5 * mask_c, 0.0, 1.0)
    adj = (xc >= 0.0).astype(_F32) * mask_c
    v_ref[...] = (cont_adj > 0.0).astype(_F32)

    t = t_ref[0, 0, 0]
    half = NF // 2
    freqs = jnp.exp(
        jax.lax.broadcasted_iota(jnp.int32, (1, half), 1).astype(_F32)
        * (-np.log(10000.0) / (half - 1)))
    ang = t * freqs * 1000.0
    temb = jnp.concatenate([jnp.sin(ang), jnp.cos(ang)], axis=1)
    temb = _mm(temb, Wt1_ref[...]) + bt1_ref[...]
    temb = _mm(_silu(temb), Wt2_ref[...]) + bt2_ref[...]
    temb_a = _silu(temb)

    eye = (jax.lax.broadcasted_iota(jnp.int32, (N, N), 0)
           == jax.lax.broadcasted_iota(jnp.int32, (N, N), 1))
    eyef = eye.astype(_F32)
    deg_b = jnp.sum(adj, axis=1, keepdims=True)
    T = adj / jnp.maximum(deg_b, 1.0)

    Pk = T
    Akf = (adj > 0.0).astype(_F32)
    reached = eyef
    spd = jnp.where(eye, 0.0, float(RW))
    hpos = jnp.zeros((N, HE), _F32)
    for k in range(1, RW + 1):
        d = jnp.sum(Pk * eyef, axis=1, keepdims=True)
        hpos = hpos + d * Wpos_ref[k - 1:k, :]
        newf = Akf * (1.0 - reached)
        spd = jnp.where(newf > 0.0, float(k), spd)
        reached = reached + newf
        if k < RW:
            Pk = _mmx(Pk, T)
            Akf = (_mm(Akf, adj) > 0.0).astype(_F32)
    hpos = hpos + bpos_ref[...]

    degc = jnp.clip(jnp.sum(cont_adj, axis=1, keepdims=True), 0.0, float(DMAX))
    degi = jnp.floor(degc)
    iota_d = jax.lax.broadcasted_iota(jnp.int32, (N, DMAX + 1), 1).astype(_F32)
    oh = (degi == iota_d).astype(_F32)
    hdeg = _mm(oh, Wdeg_ref[...]) + bdeg_ref[...]
    h = jnp.concatenate([hdeg, hpos], axis=1)

    wori = WoriT_ref[...]
    ebias = eb_ref[...]
    Wcat = jnp.concatenate(
        [wori, WspdT_ref[...], ebias, jnp.zeros((GF - RW - 3, HE), _F32)],
        axis=0)

    x_s[...] = xc
    sp_s[...] = spd

    def g_block(p, _):
        xb = x_s[pl.ds(p * P, P), :]
        sb = sp_s[pl.ds(p * P, P), :]
        vb = v_ref[pl.ds(p * P, P), :]
        iota = jax.lax.broadcasted_iota(jnp.int32, (P, N, GF), 2)
        g = (sb[:, :, None] == (iota - 1).astype(_F32)).astype(_F32)
        g = g + jnp.where(iota == 0, xb[:, :, None], 0.0)
        g = g + (iota == RW + 2).astype(_F32)
        g = g + (iota == RW + 3).astype(_F32) * (1.0 - vb[:, :, None])
        eattr_ref[pl.ds(p * P * N, P * N), :] = g.reshape(P * N, GF)
        return 0

    jax.lax.fori_loop(0, NB, g_block, 0)
    mrow = jax.lax.broadcasted_iota(jnp.int32, (GF, DN), 0)

    for l in range(L):
        s_ref[...] = _mm(h, Ws_ref[l])
        b_ref[...] = _mm(h, Wd_ref[l]) + _mm(temb_a, Wtb_ref[l])
        acc_ref[...] = jnp.zeros((N, DN), _F32)
        Ml = jnp.where(mrow == RW + 3, -1e5, _mmx(Wcat, We_ref[l]))

        def msg_block(p, _, Ml=Ml):
            gb = eattr_ref[pl.ds(p * P * N, P * N), :]
            ew = _mm(gb, Ml).reshape(P, N, DN)
            Sb = s_ref[pl.ds(p * P, P), :]
            msg = _silu(ew + Sb[:, None, :] + b_ref[...][None])
            acc_ref[...] += jnp.sum(msg, axis=0)
            return 0

        jax.lax.fori_loop(0, NB, msg_block, 0)
        h = _silu(h + _mm(acc_ref[...], Wu_ref[l]) + bu_ref[l:l + 1, :])

    a_ref[...] = _mm(h, Wpi_ref[...])
    c_ref[...] = _mm(h, Wpj_ref[...])
    bp = bp_ref[...]
    wo2 = Wo2_ref[...]
    Mpe = _mmx(Wcat, Wpe_ref[...])
    Wo1 = Wo1_ref[...]
    bo1 = bo1_ref[...]
    bo2 = bo2_ref[0, 0]

    def head_block(p, _):
        gb = eattr_ref[pl.ds(p * P * N, P * N), :]
        ep = _mm(gb, Mpe).reshape(P, N, HE)
        ab = a_ref[pl.ds(p * P, P), :]
        ed = _silu(ep + ab[:, None, :] + c_ref[...][None] + bp[None])
        v = _silu(ed).reshape(P * N, HE)
        v2 = jax.lax.dot_general(v, Wo1, (((1,), (1,)), ((), ())),
                                 preferred_element_type=_F32) + bo1
        v2 = _silu(v2).reshape(P, N, HE)
        o = jnp.sum(v2 * wo2[None], axis=2) + bo2
        out_ref[0, 0, pl.ds(p * P, P), :] = o
        return 0

    jax.lax.fori_loop(0, NB, head_block, 0)
    out = out_ref[0, 0]
    out_ref[0, 0] = (out + out.T) * 0.5 * mask_c


def _full_spec(shape):
    nd = len(shape)
    return pl.BlockSpec(shape, lambda b: (0,) * nd)


@jax.jit
def kernel(x, time_cond, mask, Wt1, bt1, Wt2, bt2, Wori, bori, Wspd, bspd,
           Wdeg, bdeg, Wpos, bpos, Ws, Wd, We, Wu, bu, Wtb, Wpi, Wpj, Wpe,
           bp, Wo1, bo1, Wo2, bo2):
    t2 = time_cond.reshape(B, 1, 1).astype(_F32)
    args = (
        x, t2, mask,
        Wt1, bt1.reshape(1, TD), Wt2, bt2.reshape(1, TD),
        Wori.T, (bori + bspd).reshape(1, HE), Wspd.T,
        Wdeg, bdeg.reshape(1, NF), Wpos, bpos.reshape(1, HE),
        Ws, Wd, We, Wu, bu, Wtb,
        Wpi, Wpj, Wpe, bp.reshape(1, HE),
        Wo1, bo1.reshape(1, HE), Wo2, bo2.reshape(1, 1),
    )
    in_specs = [
        pl.BlockSpec((1, 1, N, N), lambda b: (b, 0, 0, 0)),
        pl.BlockSpec((1, 1, 1), lambda b: (b, 0, 0)),
        pl.BlockSpec((1, 1, N, N), lambda b: (b, 0, 0, 0)),
    ] + [_full_spec(a.shape) for a in args[3:]]
    out = pl.pallas_call(
        _kernel_body,
        grid=(B,),
        in_specs=in_specs,
        out_specs=pl.BlockSpec((1, 1, N, N), lambda b: (b, 0, 0, 0)),
        out_shape=jax.ShapeDtypeStruct((B, 1, N, N), _F32),
        scratch_shapes=[
            pltpu.VMEM((N * N, GF), _F32),
            pltpu.VMEM((N, N), _F32),
            pltpu.VMEM((N, DN), _F32),
            pltpu.VMEM((N, DN), _F32),
            pltpu.VMEM((N, DN), _F32),
            pltpu.VMEM((N, HE), _F32),
            pltpu.VMEM((N, HE), _F32),
            pltpu.VMEM((N, N), _F32),
            pltpu.VMEM((N, N), _F32),
        ],
        compiler_params=pltpu.CompilerParams(
            dimension_semantics=("parallel",)),
    )(*args)
    return out


# speedup vs baseline: 2.3823x; 1.0398x over previous
import jax
import jax.numpy as jnp
import numpy as np
from jax.experimental import pallas as pl
from jax.experimental.pallas import tpu as pltpu

B, N = 8, 128
NF = 128
RW = 16
DMAX = 64
HE = 64
DN = NF + HE
TD = 4 * NF
L = 3
P = 64
NB = N // P
GF = 32

_F32 = jnp.float32


def _silu(v):
    return v * (0.5 * jnp.tanh(0.5 * v) + 0.5)


def _mm(a, b):
    return jnp.dot(a, b, preferred_element_type=_F32)


_mmx = _mm


def _kernel_body(x_ref, t_ref, mask_ref,
                 Wt1_ref, bt1_ref, Wt2_ref, bt2_ref,
                 WoriT_ref, eb_ref, WspdT_ref,
                 Wdeg_ref, bdeg_ref, Wpos_ref, bpos_ref,
                 Ws_ref, Wd_ref, We_ref, Wu_ref, bu_ref, Wtb_ref,
                 Wpi_ref, Wpj_ref, Wpe_ref, bp_ref,
                 Wo1_ref, bo1_ref, Wo2_ref, bo2_ref,
                 out_ref,
                 eattr_ref, v_ref, s_ref, b_ref, acc_ref, a_ref, c_ref,
                 x_s, sp_s):
    x_raw = x_ref[0, 0]
    m_raw = mask_ref[0, 0]
    xc = jnp.clip(jnp.where(jnp.isnan(x_raw), 0.0, x_raw), -1.0, 1.0)
    mask_c = jnp.clip(jnp.where(jnp.isnan(m_raw), 0.0, m_raw), 0.0, 1.0)
    cont_adj = jnp.clip((xc + 1.0) * 0.5 * mask_c, 0.0, 1.0)
    adj = (xc >= 0.0).astype(_F32) * mask_c
    v_ref[...] = (cont_adj > 0.0).astype(_F32)

    t = t_ref[0, 0, 0]
    half = NF // 2
    freqs = jnp.exp(
        jax.lax.broadcasted_iota(jnp.int32, (1, half), 1).astype(_F32)
        * (-np.log(10000.0) / (half - 1)))
    ang = t * freqs * 1000.0
    temb = jnp.concatenate([jnp.sin(ang), jnp.cos(ang)], axis=1)
    temb = _mm(temb, Wt1_ref[...]) + bt1_ref[...]
    temb = _mm(_silu(temb), Wt2_ref[...]) + bt2_ref[...]
    temb_a = _silu(temb)

    eye = (jax.lax.broadcasted_iota(jnp.int32, (N, N), 0)
           == jax.lax.broadcasted_iota(jnp.int32, (N, N), 1))
    eyef = eye.astype(_F32)
    deg_b = jnp.sum(adj, axis=1, keepdims=True)
    T = adj / jnp.maximum(deg_b, 1.0)

    Pk = T
    Akf = (adj > 0.0).astype(_F32)
    reached = eyef
    spd = jnp.where(eye, 0.0, float(RW))
    hpos = jnp.zeros((N, HE), _F32)
    for k in range(1, RW + 1):
        d = jnp.sum(Pk * eyef, axis=1, keepdims=True)
        hpos = hpos + d * Wpos_ref[k - 1:k, :]
        newf = Akf * (1.0 - reached)
        spd = jnp.where(newf > 0.0, float(k), spd)
        reached = reached + newf
        if k < RW:
            Pk = _mmx(Pk, T)
            Akf = (_mm(Akf, adj) > 0.0).astype(_F32)
    hpos = hpos + bpos_ref[...]

    degc = jnp.clip(jnp.sum(cont_adj, axis=1, keepdims=True), 0.0, float(DMAX))
    degi = jnp.floor(degc)
    iota_d = jax.lax.broadcasted_iota(jnp.int32, (N, DMAX + 1), 1).astype(_F32)
    oh = (degi == iota_d).astype(_F32)
    hdeg = _mm(oh, Wdeg_ref[...]) + bdeg_ref[...]
    h = jnp.concatenate([hdeg, hpos], axis=1)

    wori = WoriT_ref[...]
    ebias = eb_ref[...]
    Wcat = jnp.concatenate(
        [wori, WspdT_ref[...], ebias, jnp.zeros((GF - RW - 3, HE), _F32)],
        axis=0)

    x_s[...] = xc
    sp_s[...] = spd

    def g_block(p, _):
        xb = x_s[pl.ds(p * P, P), :]
        sb = sp_s[pl.ds(p * P, P), :]
        vb = v_ref[pl.ds(p * P, P), :]
        iota = jax.lax.broadcasted_iota(jnp.int32, (P, N, GF), 2)
        g = (sb[:, :, None] == (iota - 1).astype(_F32)).astype(_F32)
        g = g + jnp.where(iota == 0, xb[:, :, None], 0.0)
        g = g + (iota == RW + 2).astype(_F32)
        g = g + (iota == RW + 3).astype(_F32) * (1.0 - vb[:, :, None])
        eattr_ref[pl.ds(p * P * N, P * N), :] = g.reshape(P * N, GF)
        return 0

    jax.lax.fori_loop(0, NB, g_block, 0)
    mrow = jax.lax.broadcasted_iota(jnp.int32, (GF, DN), 0)

    for l in range(L):
        s_ref[...] = _mm(h, Ws_ref[l])
        b_ref[...] = _mm(h, Wd_ref[l]) + _mm(temb_a, Wtb_ref[l])
        acc_ref[...] = jnp.zeros((N, DN), _F32)
        Ml = jnp.where(mrow == RW + 3, -1e5, _mmx(Wcat, We_ref[l]))

        def msg_block(p, _, Ml=Ml):
            gb = eattr_ref[pl.ds(p * P * N, P * N), :]
            ew = _mm(gb, Ml).reshape(P, N, DN)
            Sb = s_ref[pl.ds(p * P, P), :]
            msg = _silu(ew + Sb[:, None, :] + b_ref[...][None])
            acc_ref[...] += jnp.sum(msg, axis=0)
            return 0

        jax.lax.fori_loop(0, NB, msg_block, 0)
        h = _silu(h + _mm(acc_ref[...], Wu_ref[l]) + bu_ref[l:l + 1, :])

    a_ref[...] = _mm(h, Wpi_ref[...])
    c_ref[...] = _mm(h, Wpj_ref[...])
    bp = bp_ref[...]
    wo2 = Wo2_ref[...]
    Mpe = _mmx(Wcat, Wpe_ref[...])
    Wo1 = Wo1_ref[...]
    bo1 = bo1_ref[...]
    bo2 = bo2_ref[0, 0]

    def head_block(p, _):
        gb = eattr_ref[pl.ds(p * P * N, P * N), :]
        ep = _mm(gb, Mpe).reshape(P, N, HE)
        ab = a_ref[pl.ds(p * P, P), :]
        ed = _silu(ep + ab[:, None, :] + c_ref[...][None] + bp[None])
        v = _silu(ed).reshape(P * N, HE)
        v2 = jax.lax.dot_general(v, Wo1, (((1,), (1,)), ((), ())),
                                 preferred_element_type=_F32) + bo1
        v2 = _silu(v2).reshape(P, N, HE)
        o = jnp.sum(v2 * wo2[None], axis=2) + bo2
        out_ref[0, 0, pl.ds(p * P, P), :] = o
        return 0

    jax.lax.fori_loop(0, NB, head_block, 0)
    out = out_ref[0, 0]
    out_ref[0, 0] = (out + out.T) * 0.5 * mask_c


def _full_spec(shape):
    nd = len(shape)
    return pl.BlockSpec(shape, lambda b: (0,) * nd)


@jax.jit
def kernel(x, time_cond, mask, Wt1, bt1, Wt2, bt2, Wori, bori, Wspd, bspd,
           Wdeg, bdeg, Wpos, bpos, Ws, Wd, We, Wu, bu, Wtb, Wpi, Wpj, Wpe,
           bp, Wo1, bo1, Wo2, bo2):
    t2 = time_cond.reshape(B, 1, 1).astype(_F32)
    args = (
        x, t2, mask,
        Wt1, bt1.reshape(1, TD), Wt2, bt2.reshape(1, TD),
        Wori.T, (bori + bspd).reshape(1, HE), Wspd.T,
        Wdeg, bdeg.reshape(1, NF), Wpos, bpos.reshape(1, HE),
        Ws, Wd, We, Wu, bu, Wtb,
        Wpi, Wpj, Wpe, bp.reshape(1, HE),
        Wo1, bo1.reshape(1, HE), Wo2, bo2.reshape(1, 1),
    )
    in_specs = [
        pl.BlockSpec((1, 1, N, N), lambda b: (b, 0, 0, 0)),
        pl.BlockSpec((1, 1, 1), lambda b: (b, 0, 0)),
        pl.BlockSpec((1, 1, N, N), lambda b: (b, 0, 0, 0)),
    ] + [_full_spec(a.shape) for a in args[3:]]
    out = pl.pallas_call(
        _kernel_body,
        grid=(B,),
        in_specs=in_specs,
        out_specs=pl.BlockSpec((1, 1, N, N), lambda b: (b, 0, 0, 0)),
        out_shape=jax.ShapeDtypeStruct((B, 1, N, N), _F32),
        scratch_shapes=[
            pltpu.VMEM((N * N, GF), _F32),
            pltpu.VMEM((N, N), _F32),
            pltpu.VMEM((N, DN), _F32),
            pltpu.VMEM((N, DN), _F32),
            pltpu.VMEM((N, DN), _F32),
            pltpu.VMEM((N, HE), _F32),
            pltpu.VMEM((N, HE), _F32),
            pltpu.VMEM((N, N), _F32),
            pltpu.VMEM((N, N), _F32),
        ],
        compiler_params=pltpu.CompilerParams(
            dimension_semantics=("parallel",)),
    )(*args)
    return out
